# baseline (speedup 1.0000x reference)
import functools

import jax
import jax.numpy as jnp
from jax import lax
from jax.experimental import pallas as pl
from jax.experimental.pallas import tpu as pltpu
from jax.experimental.pallas import tpu_sc as plsc

B, C, H, W = 8, 768, 32, 32
HW = H * W
NPIX = B * HW
NREF = 4
LANES = 16
NC, NS = 2, 16
NW = NC * NS
PPW = NPIX // NW
GROUP = 8
GPW = PPW // GROUP
SCALE = 1.0 / float(C) ** 0.5
NT = (((1,), (1,)), ((), ()))


def _tc1_body(x_ref, wq_ref, wk_ref, wo_ref, bq_ref, bk_ref, bo_ref,
              s_ref, gidx_ref):
    b = pl.program_id(0)
    xt = x_ref[0]
    q = lax.dot_general(xt, wq_ref[...], NT,
                        preferred_element_type=jnp.float32) + bq_ref[...]
    k = lax.dot_general(xt.astype(jnp.bfloat16),
                        wk_ref[...].astype(jnp.bfloat16), NT,
                        preferred_element_type=jnp.float32) + bk_ref[...]
    s = lax.dot_general(q.astype(jnp.bfloat16), k.astype(jnp.bfloat16),
                        NT, preferred_element_type=jnp.float32)
    for c in range(HW // 128):
        s_ref[c] = s[:, c * 128:(c + 1) * 128]
    off = lax.dot_general(wo_ref[...], q, NT,
                          preferred_element_type=jnp.float32) + bo_ref[...]
    p = lax.broadcasted_iota(jnp.int32, (1, HW), 1)
    ypix = (p // W).astype(jnp.float32)
    xpix = (p % W).astype(jnp.float32)
    for r in range(NREF):
        rx = jnp.floor(jnp.clip(xpix + off[2 * r:2 * r + 1, :], 0.0, W - 1.0))
        ry = jnp.floor(jnp.clip(ypix + off[2 * r + 1:2 * r + 2, :], 0.0, H - 1.0))
        gidx_ref[r:r + 1, :] = (b * HW + ry.astype(jnp.int32) * W
                                + rx.astype(jnp.int32))


_tc1_call = pl.pallas_call(
    _tc1_body,
    grid=(B,),
    in_specs=[
        pl.BlockSpec((1, HW, C), lambda i: (i, 0, 0)),
        pl.BlockSpec((C, C), lambda i: (0, 0)),
        pl.BlockSpec((C, C), lambda i: (0, 0)),
        pl.BlockSpec((2 * NREF, C), lambda i: (0, 0)),
        pl.BlockSpec((1, C), lambda i: (0, 0)),
        pl.BlockSpec((1, C), lambda i: (0, 0)),
        pl.BlockSpec((2 * NREF, 1), lambda i: (0, 0)),
    ],
    out_specs=[
        pl.BlockSpec((HW // 128, HW, 128), lambda i: (0, i, 0)),
        pl.BlockSpec((NREF, HW), lambda i: (0, i)),
    ],
    out_shape=[
        jax.ShapeDtypeStruct((HW // 128, NPIX, 128), jnp.float32),
        jax.ShapeDtypeStruct((NREF, NPIX), jnp.int32),
    ],
)


def _lane_perm(vec, perm):
    return lax.gather(
        vec, perm[:, None],
        lax.GatherDimensionNumbers(offset_dims=(), collapsed_slice_dims=(0,),
                                   start_index_map=(0,)),
        slice_sizes=(1,), mode=lax.GatherScatterMode.PROMISE_IN_BOUNDS)


def _lane_splat(vec, lane):
    return _lane_perm(vec, jnp.broadcast_to(lane, (LANES,)))


def _sc_body(s2, gidxt, attz, idx_all, sidx_v, s_v, att_all, sem_in):
    wid = lax.axis_index("s") * NC + lax.axis_index("c")
    g0 = wid * GPW
    pltpu.sync_copy(gidxt.at[:, pl.ds(wid * PPW, PPW)], idx_all)

    lane_iota = lax.iota(jnp.int32, LANES)

    def issue(grp, g, b):
        base = grp * GROUP
        coff = pl.multiple_of((g >> 1) * LANES, LANES)
        perm = (lane_iota & (GROUP - 1)) + (g & 1) * GROUP
        prow = jnp.broadcast_to(base, (LANES,)) + (lane_iota & (GROUP - 1))
        lo = lane_iota < GROUP
        for half in range(2):
            ca = _lane_perm(idx_all[2 * half, pl.ds(coff, LANES)], perm)
            cb = _lane_perm(idx_all[2 * half + 1, pl.ds(coff, LANES)], perm)
            li = jnp.where(lo, ca, cb) & (HW - 1)
            sidx_v[b, pl.ds(half * LANES, LANES)] = (
                ((li >> 7) << 13) + prow)
        pltpu.async_copy(s2.at[sidx_v.at[b]], s_v.at[b], sem_in)

    def wait_in(b):
        pltpu.make_async_copy(s2.at[sidx_v.at[b]], s_v.at[b], sem_in).wait()

    issue(g0, 0, 0)

    def pair(gp, _):
        acc = [jnp.zeros((LANES,), jnp.float32) for _ in range(NREF)]
        coff = pl.multiple_of(gp * LANES, LANES)
        cv = [idx_all[r, pl.ds(coff, LANES)] for r in range(NREF)]
        for b in range(2):
            g = gp * 2 + b
            grp = g0 + g
            wait_in(b)

            @pl.when(g + 1 < GPW)
            def _():
                issue(grp + 1, g + 1, 1 - b)

            for p in range(GROUP):
                lane = b * GROUP + p
                for r in range(NREF):
                    li = cv[r][lane] & (HW - 1)
                    j = (r >> 1) * LANES + (r & 1) * GROUP + p
                    start = pl.multiple_of(((li >> 4) & 7) * LANES, LANES)
                    cvec = s_v[b, j, pl.ds(start, LANES)]
                    zv = _lane_splat(cvec, li & (LANES - 1))
                    acc[r] = jnp.where(lane_iota == lane, zv, acc[r])
        aoff = pl.multiple_of(gp * LANES, LANES)
        for r in range(NREF):
            att_all[r, pl.ds(aoff, LANES)] = acc[r]
        return 0

    lax.fori_loop(0, GPW // 2, pair, 0)
    pltpu.sync_copy(att_all, attz.at[:, pl.ds(wid * PPW, PPW)])


@functools.cache
def _sc_call():
    return pl.kernel(
        _sc_body,
        out_type=jax.ShapeDtypeStruct((NREF, NPIX), jnp.float32),
        mesh=plsc.VectorSubcoreMesh(core_axis_name="c", subcore_axis_name="s"),
        scratch_types=[
            pltpu.VMEM((NREF, PPW), jnp.int32),
            pltpu.VMEM((2, GROUP * NREF), jnp.int32),
            pltpu.VMEM((2, GROUP * NREF, 128), jnp.float32),
            pltpu.VMEM((NREF, PPW), jnp.float32),
            pltpu.SemaphoreType.DMA,
        ],
    )


def _tc2_body(x_ref, wv_ref, bv_ref, gidx_ref, attz_ref, out_ref):
    vp = lax.dot_general(x_ref[0].astype(jnp.bfloat16),
                         wv_ref[...].astype(jnp.bfloat16), NT,
                         preferred_element_type=jnp.float32) + bv_ref[...]
    att = 1.0 / (1.0 + jnp.exp(-attz_ref[...] * SCALE))
    lidx_t = lax.transpose(gidx_ref[...] & (HW - 1), (1, 0))
    att_t = lax.transpose(att, (1, 0))
    iota_j = lax.broadcasted_iota(jnp.int32, (1, HW), 1)
    terms = [jnp.where(lidx_t[:, r:r + 1] == iota_j,
                       att_t[:, r:r + 1], 0.0).astype(jnp.bfloat16)
             for r in range(NREF)]
    pmat = (terms[0] + terms[1]) + (terms[2] + terms[3])
    out_ref[0] = lax.dot_general(pmat,
                                 vp.astype(jnp.bfloat16),
                                 (((1,), (0,)), ((), ())),
                                 preferred_element_type=jnp.float32)


_tc2_call = pl.pallas_call(
    _tc2_body,
    grid=(B,),
    in_specs=[
        pl.BlockSpec((1, HW, C), lambda i: (i, 0, 0)),
        pl.BlockSpec((C, C), lambda i: (0, 0)),
        pl.BlockSpec((1, C), lambda i: (0, 0)),
        pl.BlockSpec((NREF, HW), lambda i: (0, i)),
        pl.BlockSpec((NREF, HW), lambda i: (0, i)),
    ],
    out_specs=pl.BlockSpec((1, HW, C), lambda i: (i, 0, 0)),
    out_shape=jax.ShapeDtypeStruct((B, HW, C), jnp.float32),
)


def kernel(x, Wq, bq, Wk, bk, Wv, bv, Wo, bo):
    x_pm = x.transpose(0, 2, 3, 1).reshape(B, HW, C)
    s2, gidxt = _tc1_call(x_pm, Wq, Wk, Wo, bq[None, :], bk[None, :],
                          bo[:, None])
    attz = _sc_call()(s2.reshape((HW // 128) * NPIX, 128), gidxt)
    out_pm = _tc2_call(x_pm, Wv, bv[None, :], gidxt, attz)
    return out_pm.reshape(B, H, W, C).transpose(0, 3, 1, 2)


# speedup vs baseline: 1.0181x; 1.0181x over previous
import functools

import jax
import jax.numpy as jnp
from jax import lax
from jax.experimental import pallas as pl
from jax.experimental.pallas import tpu as pltpu
from jax.experimental.pallas import tpu_sc as plsc

B, C, H, W = 8, 768, 32, 32
HW = H * W
NPIX = B * HW
NREF = 4
LANES = 16
NC, NS = 2, 16
NW = NC * NS
PPW = NPIX // NW
GROUP = 8
GPW = PPW // GROUP
SCALE = 1.0 / float(C) ** 0.5
NT = (((1,), (1,)), ((), ()))


def _tc1_body(x_ref, wq_ref, wk_ref, wo_ref, bq_ref, bk_ref, bo_ref,
              s_ref, gidx_ref):
    b = pl.program_id(0)
    xt = x_ref[0]
    q = lax.dot_general(xt, wq_ref[...], NT,
                        preferred_element_type=jnp.float32) + bq_ref[...]
    k = lax.dot_general(xt.astype(jnp.bfloat16),
                        wk_ref[...].astype(jnp.bfloat16), NT,
                        preferred_element_type=jnp.float32) + bk_ref[...]
    s = lax.dot_general(q.astype(jnp.bfloat16), k.astype(jnp.bfloat16),
                        NT, preferred_element_type=jnp.float32)
    for c in range(HW // 128):
        s_ref[c] = s[:, c * 128:(c + 1) * 128]
    off = lax.dot_general(wo_ref[...], q, NT,
                          preferred_element_type=jnp.float32) + bo_ref[...]
    p = lax.broadcasted_iota(jnp.int32, (1, HW), 1)
    ypix = (p // W).astype(jnp.float32)
    xpix = (p % W).astype(jnp.float32)
    for r in range(NREF):
        rx = jnp.floor(jnp.clip(xpix + off[2 * r:2 * r + 1, :], 0.0, W - 1.0))
        ry = jnp.floor(jnp.clip(ypix + off[2 * r + 1:2 * r + 2, :], 0.0, H - 1.0))
        gidx_ref[r:r + 1, :] = (b * HW + ry.astype(jnp.int32) * W
                                + rx.astype(jnp.int32))


_tc1_call = pl.pallas_call(
    _tc1_body,
    grid=(B,),
    in_specs=[
        pl.BlockSpec((1, HW, C), lambda i: (i, 0, 0)),
        pl.BlockSpec((C, C), lambda i: (0, 0)),
        pl.BlockSpec((C, C), lambda i: (0, 0)),
        pl.BlockSpec((2 * NREF, C), lambda i: (0, 0)),
        pl.BlockSpec((1, C), lambda i: (0, 0)),
        pl.BlockSpec((1, C), lambda i: (0, 0)),
        pl.BlockSpec((2 * NREF, 1), lambda i: (0, 0)),
    ],
    out_specs=[
        pl.BlockSpec((HW // 128, HW, 128), lambda i: (0, i, 0)),
        pl.BlockSpec((NREF, HW), lambda i: (0, i)),
    ],
    out_shape=[
        jax.ShapeDtypeStruct((HW // 128, NPIX, 128), jnp.float32),
        jax.ShapeDtypeStruct((NREF, NPIX), jnp.int32),
    ],
)


def _lane_perm(vec, perm):
    return lax.gather(
        vec, perm[:, None],
        lax.GatherDimensionNumbers(offset_dims=(), collapsed_slice_dims=(0,),
                                   start_index_map=(0,)),
        slice_sizes=(1,), mode=lax.GatherScatterMode.PROMISE_IN_BOUNDS)


def _lane_splat(vec, lane):
    return _lane_perm(vec, jnp.broadcast_to(lane, (LANES,)))


def _sc_body(s2, gidxt, attz, idx_all, sidx_v, s_v, att_all, sem_in):
    wid = lax.axis_index("s") * NC + lax.axis_index("c")
    g0 = wid * GPW
    pltpu.sync_copy(gidxt.at[:, pl.ds(wid * PPW, PPW)], idx_all)

    lane_iota = lax.iota(jnp.int32, LANES)

    def issue(grp, g, b):
        base = grp * GROUP
        coff = pl.multiple_of((g >> 1) * LANES, LANES)
        perm = (lane_iota & (GROUP - 1)) + (g & 1) * GROUP
        prow = jnp.broadcast_to(base, (LANES,)) + (lane_iota & (GROUP - 1))
        lo = lane_iota < GROUP
        for half in range(2):
            ca = _lane_perm(idx_all[2 * half, pl.ds(coff, LANES)], perm)
            cb = _lane_perm(idx_all[2 * half + 1, pl.ds(coff, LANES)], perm)
            li = jnp.where(lo, ca, cb) & (HW - 1)
            sidx_v[b, pl.ds(half * LANES, LANES)] = (
                ((li >> 7) << 13) + prow)
        pltpu.async_copy(s2.at[sidx_v.at[b]], s_v.at[b], sem_in)

    def wait_in(b):
        pltpu.make_async_copy(s2.at[sidx_v.at[b]], s_v.at[b], sem_in).wait()

    issue(g0, 0, 0)

    def pair(gp, _):
        acc = [jnp.zeros((LANES,), jnp.float32) for _ in range(NREF)]
        coff = pl.multiple_of(gp * LANES, LANES)
        cv = [idx_all[r, pl.ds(coff, LANES)] for r in range(NREF)]
        for b in range(2):
            g = gp * 2 + b
            grp = g0 + g
            wait_in(b)

            @pl.when(g + 1 < GPW)
            def _():
                issue(grp + 1, g + 1, 1 - b)

            for p in range(GROUP):
                lane = b * GROUP + p
                for r in range(NREF):
                    li = cv[r][lane] & (HW - 1)
                    j = (r >> 1) * LANES + (r & 1) * GROUP + p
                    start = pl.multiple_of(((li >> 4) & 7) * LANES, LANES)
                    cvec = s_v[b, j, pl.ds(start, LANES)]
                    zv = _lane_splat(cvec, li & (LANES - 1))
                    acc[r] = jnp.where(lane_iota == lane, zv, acc[r])
        aoff = pl.multiple_of(gp * LANES, LANES)
        for r in range(NREF):
            att_all[r, pl.ds(aoff, LANES)] = acc[r]
        return 0

    lax.fori_loop(0, GPW // 2, pair, 0)
    pltpu.sync_copy(att_all, attz.at[:, pl.ds(wid * PPW, PPW)])


@functools.cache
def _sc_call():
    return pl.kernel(
        _sc_body,
        out_type=jax.ShapeDtypeStruct((NREF, NPIX), jnp.float32),
        mesh=plsc.VectorSubcoreMesh(core_axis_name="c", subcore_axis_name="s"),
        scratch_types=[
            pltpu.VMEM((NREF, PPW), jnp.int32),
            pltpu.VMEM((2, GROUP * NREF), jnp.int32),
            pltpu.VMEM((2, GROUP * NREF, 128), jnp.float32),
            pltpu.VMEM((NREF, PPW), jnp.float32),
            pltpu.SemaphoreType.DMA,
        ],
    )


def _tc2_body(x_ref, wv_ref, bv_ref, gidx_ref, attz_ref, out_ref):
    vp = lax.dot_general(x_ref[0].astype(jnp.bfloat16),
                         wv_ref[...].astype(jnp.bfloat16), NT,
                         preferred_element_type=jnp.float32) + bv_ref[...]
    att = 1.0 / (1.0 + jnp.exp(-attz_ref[...] * SCALE))
    lidx = gidx_ref[...] & (HW - 1)
    iota_i = lax.broadcasted_iota(jnp.int32, (HW, 1), 0)
    terms = [jnp.where(iota_i == lidx[r:r + 1, :],
                       att[r:r + 1, :], 0.0).astype(jnp.bfloat16)
             for r in range(NREF)]
    pmat_t = (terms[0] + terms[1]) + (terms[2] + terms[3])
    out_ref[0] = lax.dot_general(pmat_t,
                                 vp.astype(jnp.bfloat16),
                                 (((0,), (0,)), ((), ())),
                                 preferred_element_type=jnp.float32)


_tc2_call = pl.pallas_call(
    _tc2_body,
    grid=(B,),
    in_specs=[
        pl.BlockSpec((1, HW, C), lambda i: (i, 0, 0)),
        pl.BlockSpec((C, C), lambda i: (0, 0)),
        pl.BlockSpec((1, C), lambda i: (0, 0)),
        pl.BlockSpec((NREF, HW), lambda i: (0, i)),
        pl.BlockSpec((NREF, HW), lambda i: (0, i)),
    ],
    out_specs=pl.BlockSpec((1, HW, C), lambda i: (i, 0, 0)),
    out_shape=jax.ShapeDtypeStruct((B, HW, C), jnp.float32),
)


def kernel(x, Wq, bq, Wk, bk, Wv, bv, Wo, bo):
    x_pm = x.transpose(0, 2, 3, 1).reshape(B, HW, C)
    s2, gidxt = _tc1_call(x_pm, Wq, Wk, Wo, bq[None, :], bk[None, :],
                          bo[:, None])
    attz = _sc_call()(s2.reshape((HW // 128) * NPIX, 128), gidxt)
    out_pm = _tc2_call(x_pm, Wv, bv[None, :], gidxt, attz)
    return out_pm.reshape(B, H, W, C).transpose(0, 3, 1, 2)
